# baseline (speedup 1.0000x reference)
import numpy as np

import jax
import jax.numpy as jnp
from jax import lax
from jax.experimental import pallas as pl
from jax.experimental.pallas import tpu as pltpu
from jax.experimental.pallas import tpu_sc as plsc

N_PTS = 262144
RES = 512
ODIM = 32
NC, NS = 2, 16
NW = NC * NS
PTS_PER_W = N_PTS // NW
P = 64
N_CHUNKS = PTS_PER_W // P

_MAT_IDS = ((0, 1), (0, 2), (1, 2))
_VEC_IDS = (2, 1, 0)


def _prep_coord(c):
    p = c * (RES / 2.0) + (RES / 2.0 - 0.5)
    fi = p.astype(jnp.int32)
    fi = jnp.where(fi.astype(jnp.float32) > p, fi - 1, fi)
    t = p - fi.astype(jnp.float32)
    i0 = jnp.clip(fi, 0, RES - 1)
    i1 = jnp.clip(fi + 1, 0, RES - 1)
    v0 = ((fi >= 0) & (fi <= RES - 1)).astype(jnp.float32)
    v1 = ((fi >= -1) & (fi <= RES - 2)).astype(jnp.float32)
    w0 = (1.0 - t) * v0
    w1 = t * v1
    return i0, i1, w0, w1


def _body(xp_hbm, mat_hbm, vec_hbm, out_hbm,
          xch, widx, wallT, rows, outbuf, sem):
    wid = lax.axis_index("s") * NC + lax.axis_index("c")
    lanes = lax.iota(jnp.int32, 16)
    zeros = lanes * 0

    def chunk_body(c, _):
        base = pl.multiple_of(wid * PTS_PER_W + c * P, P)

        pltpu.sync_copy(xp_hbm.at[wid * N_CHUNKS + c], xch)

        for g in range(P // 16):
            sl = pl.ds(g * 16, 16)
            pre = [_prep_coord(xch[j, sl]) for j in range(3)]
            for i in range(3):
                a, b = _MAT_IDS[i]
                xi0, xi1, wx0, wx1 = pre[a]
                yi0, yi1, wy0, wy1 = pre[b]
                pbase = i * (RES * RES)
                r0 = pbase + yi0 * RES
                r1 = pbase + yi1 * RES
                widx[4 * i + 0, sl] = r0 + xi0
                widx[4 * i + 1, sl] = r0 + xi1
                widx[4 * i + 2, sl] = r1 + xi0
                widx[4 * i + 3, sl] = r1 + xi1
                wallT[4 * i + 0, sl] = wy0 * wx0
                wallT[4 * i + 1, sl] = wy0 * wx1
                wallT[4 * i + 2, sl] = wy1 * wx0
                wallT[4 * i + 3, sl] = wy1 * wx1
                zi0, zi1, wz0, wz1 = pre[_VEC_IDS[i]]
                widx[12 + 2 * i + 0, sl] = i * RES + zi0
                widx[12 + 2 * i + 1, sl] = i * RES + zi1
                wallT[12 + 2 * i + 0, sl] = wz0
                wallT[12 + 2 * i + 1, sl] = wz1

        descs = [pltpu.async_copy(mat_hbm.at[widx.at[j]],
                                  rows.at[pl.ds(j * P, P)], sem)
                 for j in range(12)]
        descs += [pltpu.async_copy(vec_hbm.at[widx.at[12 + j]],
                                   rows.at[pl.ds((12 + j) * P, P)], sem)
                  for j in range(6)]
        for d in descs:
            d.wait()

        def group_body(g, carry):
            sl = pl.ds(g * 16, 16)
            pb = g * 16 + lanes
            w = [wallT[j, sl] for j in range(18)]
            rb = [pb + j * P for j in range(18)]

            def ch(j, cc):
                return plsc.load_gather(rows, [rb[j], cc])

            for i in range(3):
                for c in range(ODIM):
                    cc = zeros + c
                    acc = w[4 * i + 0] * ch(4 * i + 0, cc)
                    acc = acc + w[4 * i + 1] * ch(4 * i + 1, cc)
                    acc = acc + w[4 * i + 2] * ch(4 * i + 2, cc)
                    acc = acc + w[4 * i + 3] * ch(4 * i + 3, cc)
                    v = (w[12 + 2 * i] * ch(12 + 2 * i, cc)
                         + w[13 + 2 * i] * ch(13 + 2 * i, cc))
                    plsc.store_scatter(
                        outbuf, [pb, zeros + (ODIM * i + c)], acc * v)
            return carry

        lax.fori_loop(0, P // 16, group_body, None)

        pltpu.sync_copy(outbuf, out_hbm.at[pl.ds(base, P)])
        return None

    lax.fori_loop(0, N_CHUNKS, chunk_body, None)


@jax.jit
def _encode(xp, mat_tab, vec_tab):
    mesh = plsc.VectorSubcoreMesh(core_axis_name="c", subcore_axis_name="s",
                                  num_cores=NC, num_subcores=NS)
    run = pl.kernel(
        _body,
        out_type=jax.ShapeDtypeStruct((N_PTS, 3 * ODIM), jnp.float32),
        mesh=mesh,
        compiler_params=pltpu.CompilerParams(
            use_tc_tiling_on_sc=False, needs_layout_passes=False),
        scratch_types=[
            pltpu.VMEM((3, P), jnp.float32),
            pltpu.VMEM((18, P), jnp.int32),
            pltpu.VMEM((18, P), jnp.float32),
            pltpu.VMEM((18 * P, ODIM), jnp.float32),
            pltpu.VMEM((P, 3 * ODIM), jnp.float32),
            pltpu.SemaphoreType.DMA,
        ],
    )
    return run(xp, mat_tab, vec_tab)


def kernel(x, C_mat, C_vec):
    mat_tab = jnp.transpose(C_mat, (0, 2, 3, 1)).reshape(3 * RES * RES, ODIM)
    vec_tab = jnp.transpose(C_vec[:, :, :, 0], (0, 2, 1)).reshape(3 * RES, ODIM)
    xp = jnp.transpose(x.T.reshape(3, NW * N_CHUNKS, P), (1, 0, 2))
    return _encode(xp, mat_tab, vec_tab)


# speedup vs baseline: 1.0716x; 1.0716x over previous
import numpy as np

import jax
import jax.numpy as jnp
from jax import lax
from jax.experimental import pallas as pl
from jax.experimental.pallas import tpu as pltpu
from jax.experimental.pallas import tpu_sc as plsc

N_PTS = 262144
RES = 512
ODIM = 32
NC, NS = 2, 16
NW = NC * NS
PTS_PER_W = N_PTS // NW
P = 64
N_CHUNKS = PTS_PER_W // P
RB = 18 * P

_MAT_IDS = ((0, 1), (0, 2), (1, 2))
_VEC_IDS = (2, 1, 0)


def _prep_coord(c):
    p = c * (RES / 2.0) + (RES / 2.0 - 0.5)
    fi = p.astype(jnp.int32)
    fi = jnp.where(fi.astype(jnp.float32) > p, fi - 1, fi)
    t = p - fi.astype(jnp.float32)
    i0 = jnp.clip(fi, 0, RES - 1)
    i1 = jnp.clip(fi + 1, 0, RES - 1)
    v0 = ((fi >= 0) & (fi <= RES - 1)).astype(jnp.float32)
    v1 = ((fi >= -1) & (fi <= RES - 2)).astype(jnp.float32)
    w0 = (1.0 - t) * v0
    w1 = t * v1
    return i0, i1, w0, w1


def _body(xp_hbm, mat_hbm, vec_hbm, out_hbm,
          xall, widx, wallT, rows, outbuf, gsem):
    wid = lax.axis_index("s") * NC + lax.axis_index("c")
    lanes = lax.iota(jnp.int32, 16)
    zeros = lanes * 0

    pltpu.sync_copy(xp_hbm.at[wid], xall)

    def gather_descs(buf):
        ds = [pltpu.make_async_copy(mat_hbm.at[widx.at[buf, j]],
                                    rows.at[pl.ds(buf * RB + j * P, P)],
                                    gsem.at[buf])
              for j in range(12)]
        ds += [pltpu.make_async_copy(vec_hbm.at[widx.at[buf, 12 + j]],
                                     rows.at[pl.ds(buf * RB + (12 + j) * P, P)],
                                     gsem.at[buf])
               for j in range(6)]
        return ds

    def fire(c, buf):
        for g in range(P // 16):
            sl = pl.ds(g * 16, 16)
            pre = [_prep_coord(xall[c * 3 + j, sl]) for j in range(3)]
            for i in range(3):
                a, b = _MAT_IDS[i]
                xi0, xi1, wx0, wx1 = pre[a]
                yi0, yi1, wy0, wy1 = pre[b]
                pbase = i * (RES * RES)
                r0 = pbase + yi0 * RES
                r1 = pbase + yi1 * RES
                widx[buf, 4 * i + 0, sl] = r0 + xi0
                widx[buf, 4 * i + 1, sl] = r0 + xi1
                widx[buf, 4 * i + 2, sl] = r1 + xi0
                widx[buf, 4 * i + 3, sl] = r1 + xi1
                wallT[buf, 4 * i + 0, sl] = wy0 * wx0
                wallT[buf, 4 * i + 1, sl] = wy0 * wx1
                wallT[buf, 4 * i + 2, sl] = wy1 * wx0
                wallT[buf, 4 * i + 3, sl] = wy1 * wx1
                zi0, zi1, wz0, wz1 = pre[_VEC_IDS[i]]
                widx[buf, 12 + 2 * i + 0, sl] = i * RES + zi0
                widx[buf, 12 + 2 * i + 1, sl] = i * RES + zi1
                wallT[buf, 12 + 2 * i + 0, sl] = wz0
                wallT[buf, 12 + 2 * i + 1, sl] = wz1
        for d in gather_descs(buf):
            d.start()

    def process(c, buf):
        for d in gather_descs(buf):
            d.wait()
        roff = buf * RB

        def group_body(g, carry):
            sl = pl.ds(g * 16, 16)
            pb = g * 16 + lanes
            w = [wallT[buf, j, sl] for j in range(18)]
            rb = [roff + pb + j * P for j in range(18)]

            def ch(j, cc):
                return plsc.load_gather(rows, [rb[j], cc])

            for i in range(3):
                for cn in range(ODIM):
                    cc = zeros + cn
                    acc = w[4 * i + 0] * ch(4 * i + 0, cc)
                    acc = acc + w[4 * i + 1] * ch(4 * i + 1, cc)
                    acc = acc + w[4 * i + 2] * ch(4 * i + 2, cc)
                    acc = acc + w[4 * i + 3] * ch(4 * i + 3, cc)
                    v = (w[12 + 2 * i] * ch(12 + 2 * i, cc)
                         + w[13 + 2 * i] * ch(13 + 2 * i, cc))
                    plsc.store_scatter(
                        outbuf, [pb, zeros + (ODIM * i + cn)], acc * v)
            return carry

        lax.fori_loop(0, P // 16, group_body, None)

        base = pl.multiple_of(wid * PTS_PER_W, P) + c * P
        pltpu.sync_copy(outbuf, out_hbm.at[pl.ds(base, P)])

    fire(0, 0)

    def chunk_body(c, _):
        bf = c & 1
        fire(c, bf)
        process(c - 1, 1 - bf)
        return None

    lax.fori_loop(1, N_CHUNKS, chunk_body, None)
    process(N_CHUNKS - 1, (N_CHUNKS - 1) & 1)


@jax.jit
def _encode(xp, mat_tab, vec_tab):
    mesh = plsc.VectorSubcoreMesh(core_axis_name="c", subcore_axis_name="s",
                                  num_cores=NC, num_subcores=NS)
    run = pl.kernel(
        _body,
        out_type=jax.ShapeDtypeStruct((N_PTS, 3 * ODIM), jnp.float32),
        mesh=mesh,
        compiler_params=pltpu.CompilerParams(
            use_tc_tiling_on_sc=False, needs_layout_passes=False),
        scratch_types=[
            pltpu.VMEM((N_CHUNKS * 3, P), jnp.float32),
            pltpu.VMEM((2, 18, P), jnp.int32),
            pltpu.VMEM((2, 18, P), jnp.float32),
            pltpu.VMEM((2 * RB, ODIM), jnp.float32),
            pltpu.VMEM((P, 3 * ODIM), jnp.float32),
            pltpu.SemaphoreType.DMA((2,)),
        ],
    )
    return run(xp, mat_tab, vec_tab)


def kernel(x, C_mat, C_vec):
    mat_tab = jnp.transpose(C_mat, (0, 2, 3, 1)).reshape(3 * RES * RES, ODIM)
    vec_tab = jnp.transpose(C_vec[:, :, :, 0], (0, 2, 1)).reshape(3 * RES, ODIM)
    xp = jnp.transpose(x.T.reshape(3, NW * N_CHUNKS, P), (1, 0, 2))
    xp = xp.reshape(NW, N_CHUNKS * 3, P)
    return _encode(xp, mat_tab, vec_tab)


# speedup vs baseline: 1.0780x; 1.0060x over previous
import numpy as np

import jax
import jax.numpy as jnp
from jax import lax
from jax.experimental import pallas as pl
from jax.experimental.pallas import tpu as pltpu
from jax.experimental.pallas import tpu_sc as plsc

N_PTS = 262144
RES = 512
ODIM = 32
NC, NS = 2, 16
NW = NC * NS
PTS_PER_W = N_PTS // NW
P = 64
N_CHUNKS = PTS_PER_W // P
RB = 18 * P

_MAT_IDS = ((0, 1), (0, 2), (1, 2))
_VEC_IDS = (2, 1, 0)


def _prep_coord(c):
    p = c * (RES / 2.0) + (RES / 2.0 - 0.5)
    fi = p.astype(jnp.int32)
    fi = jnp.where(fi.astype(jnp.float32) > p, fi - 1, fi)
    t = p - fi.astype(jnp.float32)
    i0 = jnp.clip(fi, 0, RES - 1)
    i1 = jnp.clip(fi + 1, 0, RES - 1)
    v0 = ((fi >= 0) & (fi <= RES - 1)).astype(jnp.float32)
    v1 = ((fi >= -1) & (fi <= RES - 2)).astype(jnp.float32)
    w0 = (1.0 - t) * v0
    w1 = t * v1
    return i0, i1, w0, w1


def _body(xp_hbm, mat_hbm, vec_hbm, out_hbm,
          xall, widx, wallT, rows, outbuf, sem0, sem1):
    wid = lax.axis_index("s") * NC + lax.axis_index("c")
    lanes = lax.iota(jnp.int32, 16)
    zeros = lanes * 0
    sems = (sem0, sem1)

    pltpu.sync_copy(xp_hbm.at[wid], xall)

    def descs(b):
        ds = [pltpu.make_async_copy(mat_hbm.at[widx.at[b, j]],
                                    rows.at[pl.ds(b * RB + j * P, P)],
                                    sems[b])
              for j in range(12)]
        ds += [pltpu.make_async_copy(vec_hbm.at[widx.at[b, 12 + j]],
                                     rows.at[pl.ds(b * RB + (12 + j) * P, P)],
                                     sems[b])
               for j in range(6)]
        return ds

    def fire(c, b):
        for g in range(P // 16):
            sl = pl.ds(g * 16, 16)
            pre = [_prep_coord(xall[c * 3 + j, sl]) for j in range(3)]
            for i in range(3):
                ia, ib = _MAT_IDS[i]
                xi0, xi1, wx0, wx1 = pre[ia]
                yi0, yi1, wy0, wy1 = pre[ib]
                pbase = i * (RES * RES)
                r0 = pbase + yi0 * RES
                r1 = pbase + yi1 * RES
                widx[b, 4 * i + 0, sl] = r0 + xi0
                widx[b, 4 * i + 1, sl] = r0 + xi1
                widx[b, 4 * i + 2, sl] = r1 + xi0
                widx[b, 4 * i + 3, sl] = r1 + xi1
                wallT[b, 4 * i + 0, sl] = wy0 * wx0
                wallT[b, 4 * i + 1, sl] = wy0 * wx1
                wallT[b, 4 * i + 2, sl] = wy1 * wx0
                wallT[b, 4 * i + 3, sl] = wy1 * wx1
                zi0, zi1, wz0, wz1 = pre[_VEC_IDS[i]]
                widx[b, 12 + 2 * i + 0, sl] = i * RES + zi0
                widx[b, 12 + 2 * i + 1, sl] = i * RES + zi1
                wallT[b, 12 + 2 * i + 0, sl] = wz0
                wallT[b, 12 + 2 * i + 1, sl] = wz1
        for d in descs(b):
            d.start()

    def process(c, b):
        for d in descs(b):
            d.wait()

        def group_body(g, carry):
            sl = pl.ds(g * 16, 16)
            pb = g * 16 + lanes
            w = [wallT[b, j, sl] for j in range(18)]
            rb = [b * RB + pb + j * P for j in range(18)]

            def ch(j, cc):
                return plsc.load_gather(rows, [rb[j], cc])

            for i in range(3):
                for cn in range(ODIM):
                    cc = zeros + cn
                    acc = w[4 * i + 0] * ch(4 * i + 0, cc)
                    acc = acc + w[4 * i + 1] * ch(4 * i + 1, cc)
                    acc = acc + w[4 * i + 2] * ch(4 * i + 2, cc)
                    acc = acc + w[4 * i + 3] * ch(4 * i + 3, cc)
                    v = (w[12 + 2 * i] * ch(12 + 2 * i, cc)
                         + w[13 + 2 * i] * ch(13 + 2 * i, cc))
                    plsc.store_scatter(
                        outbuf, [pb, zeros + (ODIM * i + cn)], acc * v)
            return carry

        lax.fori_loop(0, P // 16, group_body, None)

        base = pl.multiple_of(wid * PTS_PER_W, P) + c * P
        pltpu.sync_copy(outbuf, out_hbm.at[pl.ds(base, P)])

    fire(0, 0)

    def pair_body(cp, _):
        c0 = 2 * cp
        fire(c0 + 1, 1)
        process(c0, 0)
        fire(jnp.minimum(c0 + 2, N_CHUNKS - 1), 0)
        process(c0 + 1, 1)
        return None

    lax.fori_loop(0, N_CHUNKS // 2, pair_body, None)
    for d in descs(0):
        d.wait()


@jax.jit
def _encode(xp, mat_tab, vec_tab):
    mesh = plsc.VectorSubcoreMesh(core_axis_name="c", subcore_axis_name="s",
                                  num_cores=NC, num_subcores=NS)
    run = pl.kernel(
        _body,
        out_type=jax.ShapeDtypeStruct((N_PTS, 3 * ODIM), jnp.float32),
        mesh=mesh,
        compiler_params=pltpu.CompilerParams(
            use_tc_tiling_on_sc=False, needs_layout_passes=False),
        scratch_types=[
            pltpu.VMEM((N_CHUNKS * 3, P), jnp.float32),
            pltpu.VMEM((2, 18, P), jnp.int32),
            pltpu.VMEM((2, 18, P), jnp.float32),
            pltpu.VMEM((2 * RB, ODIM), jnp.float32),
            pltpu.VMEM((P, 3 * ODIM), jnp.float32),
            pltpu.SemaphoreType.DMA,
            pltpu.SemaphoreType.DMA,
        ],
    )
    return run(xp, mat_tab, vec_tab)


def kernel(x, C_mat, C_vec):
    mat_tab = jnp.transpose(C_mat, (0, 2, 3, 1)).reshape(3 * RES * RES, ODIM)
    vec_tab = jnp.transpose(C_vec[:, :, :, 0], (0, 2, 1)).reshape(3 * RES, ODIM)
    xp = jnp.transpose(x.T.reshape(3, NW * N_CHUNKS, P), (1, 0, 2))
    xp = xp.reshape(NW, N_CHUNKS * 3, P)
    return _encode(xp, mat_tab, vec_tab)
